# baseline (speedup 1.0000x reference)
import functools

import jax
import jax.numpy as jnp
from jax import lax
from jax.experimental import pallas as pl
from jax.experimental.pallas import tpu as pltpu
from jax.experimental.pallas import tpu_sc as plsc

SCALE = 8.0
NC = 2
NS = 16
NW = NC * NS
CR = 32
NBUF = 2


@functools.lru_cache(maxsize=None)
def _make_emb(R, S, V, D):
    rpw = R // NW
    nchunk = rpw // CR
    mesh = plsc.VectorSubcoreMesh(core_axis_name="c", subcore_axis_name="s")

    @functools.partial(
        pl.kernel,
        mesh=mesh,
        compiler_params=pltpu.CompilerParams(use_tc_tiling_on_sc=False),
        out_type=jax.ShapeDtypeStruct((R, S, D), jnp.float32),
        scratch_types=[
            pltpu.VMEM((rpw, S), jnp.int32),
            pltpu.VMEM((NBUF, CR, S, D), jnp.float32),
        ]
        + [pltpu.SemaphoreType.DMA for _ in range(2 * NBUF)],
    )
    def emb(idx_hbm, table_hbm, out_hbm, idx_v, rows_v, *sems):
        gsems = sems[:NBUF]
        ssems = sems[NBUF:]
        wid = lax.axis_index("s") * NC + lax.axis_index("c")
        row0 = pl.multiple_of(wid * rpw, rpw)

        pltpu.sync_copy(idx_hbm.at[pl.ds(row0, rpw)], idx_v)

        def fire(c):
            b = c % NBUF
            return [
                pltpu.async_copy(
                    table_hbm.at[idx_v.at[c * CR + r]],
                    rows_v.at[b, r], gsems[b])
                for r in range(CR)
            ]

        def scale(b):
            def scale_body(r, c2):
                for s in range(S):
                    for j in range(D // 16):
                        sl = (b, r, s, pl.ds(j * 16, 16))
                        rows_v[sl] = rows_v[sl] * SCALE
                return c2
            lax.fori_loop(0, CR, scale_body, 0)

        ghandles = {}
        shandles = {}
        ghandles[0] = fire(0)
        for c in range(nchunk):
            b = c % NBUF
            n = c + 1
            if n < nchunk:
                if n >= NBUF:
                    shandles.pop(n - NBUF).wait()
                ghandles[n] = fire(n)
            for h in ghandles.pop(c):
                h.wait()
            scale(b)
            roff = row0 + c * CR
            shandles[c] = pltpu.async_copy(
                rows_v.at[b], out_hbm.at[pl.ds(roff, CR)], ssems[b])
        for c in sorted(shandles):
            shandles.pop(c).wait()

    return emb


def kernel(X, table):
    R, S = X.shape
    V, D = table.shape
    return _make_emb(R, S, V, D)(X.astype(jnp.int32), table)
